# Initial kernel scaffold; baseline (speedup 1.0000x reference)
import jax
import jax.numpy as jnp
from jax.experimental import pallas as pl


def kernel(verts, edges, w0a, b0a, w1a, b1a, w0b, b0b, w1b, b1b, fc1_w, fc1_b, fc2_w, fc2_b):
    raise NotImplementedError("write your pallas kernel here")


# speedup vs baseline: 5.5055x; 5.5055x over previous
import functools

import jax
import jax.numpy as jnp
from jax import lax
from jax.experimental import pallas as pl
from jax.experimental.pallas import tpu as pltpu
from jax.experimental.pallas import tpu_sc as plsc

B = 16
VPM = 5850
V = B * VPM
E = 280704

NC = 2
NS = 16
NW = NC * NS

RT = 94208
D = 16
CH = 128
EP = 282624
EPW = EP // NW
CHUNKS = EPW // CH
RPT = RT // NS

ROW_BLK = 4096
GRID1 = RT // ROW_BLK

KP = 58880
KB = 1280
GRIDF = KP // KB


def _leaky_relu(x):
    return jnp.where(x >= 0, x, 0.01 * x)


@functools.lru_cache(maxsize=None)
def _make_spmv():
    mesh = plsc.VectorSubcoreMesh(core_axis_name="c", subcore_axis_name="s")

    @functools.partial(
        pl.kernel,
        out_type=jax.ShapeDtypeStruct((NC, RT, D), jnp.float32),
        mesh=mesh,
        compiler_params=pltpu.CompilerParams(use_tc_tiling_on_sc=False),
        scratch_types=[
            pltpu.VMEM_SHARED((RT, D), jnp.float32),
            pltpu.VMEM((CH,), jnp.int32),
            pltpu.VMEM((CH,), jnp.int32),
            pltpu.VMEM((CH, D), jnp.float32),
            pltpu.VMEM((CH, D), jnp.float32),
            pltpu.SemaphoreType.DMA,
            pltpu.SemaphoreType.DMA,
        ],
    )
    def spmv(table_hbm, src_hbm, dst_hbm, zeros_hbm, out_hbm,
             acc_sp, idx_s, idx_d, rows_d, rows_s, sem1, sem2):
        c = lax.axis_index("c")
        s = lax.axis_index("s")
        w = s * NC + c

        pltpu.sync_copy(zeros_hbm.at[pl.ds(s * RPT, RPT)],
                        acc_sp.at[pl.ds(s * RPT, RPT)])
        plsc.subcore_barrier()

        def body(i, carry):
            off = w * EPW + i * CH
            pltpu.sync_copy(src_hbm.at[pl.ds(off, CH)], idx_s)
            pltpu.sync_copy(dst_hbm.at[pl.ds(off, CH)], idx_d)
            cp_d = pltpu.async_copy(table_hbm.at[idx_d], rows_d, sem1)
            cp_s = pltpu.async_copy(table_hbm.at[idx_s], rows_s, sem2)
            cp_d.wait()
            pltpu.sync_copy(rows_d, acc_sp.at[idx_s], add=True)
            cp_s.wait()
            pltpu.sync_copy(rows_s, acc_sp.at[idx_d], add=True)
            return carry

        lax.fori_loop(0, CHUNKS, body, 0)
        plsc.subcore_barrier()
        pltpu.sync_copy(acc_sp.at[pl.ds(s * RPT, RPT)],
                        out_hbm.at[c, pl.ds(s * RPT, RPT)])

    return spmv


def _dense1_body(vaug_ref, acc_ref, w1_ref, w2_ref, out_ref):
    sfull = acc_ref[0] + acc_ref[1]
    x = jnp.dot(vaug_ref[...], w1_ref[...], preferred_element_type=jnp.float32)
    x += jnp.dot(sfull, w2_ref[...], preferred_element_type=jnp.float32)
    out_ref[...] = _leaky_relu(x)


def _dense1(vaug, acc, w1c, w2c):
    return pl.pallas_call(
        _dense1_body,
        grid=(GRID1,),
        in_specs=[
            pl.BlockSpec((ROW_BLK, D), lambda i: (i, 0)),
            pl.BlockSpec((NC, ROW_BLK, D), lambda i: (0, i, 0)),
            pl.BlockSpec((D, D), lambda i: (0, 0)),
            pl.BlockSpec((D, D), lambda i: (0, 0)),
        ],
        out_specs=pl.BlockSpec((ROW_BLK, D), lambda i: (i, 0)),
        out_shape=jax.ShapeDtypeStruct((RT, D), jnp.float32),
    )(vaug, acc, w1c, w2c)


def _dense2_body(x1_ref, acc_ref, w1_ref, w2_ref, fw_ref, fb_ref, out_ref):
    sfull = acc_ref[0] + acc_ref[1]
    x2 = jnp.dot(x1_ref[...], w1_ref[...], preferred_element_type=jnp.float32)
    x2 += jnp.dot(sfull, w2_ref[...], preferred_element_type=jnp.float32)
    x2 = _leaky_relu(x2)
    x3 = jnp.dot(x2, fw_ref[...], preferred_element_type=jnp.float32)
    x3 += fb_ref[...]
    out_ref[...] = _leaky_relu(x3)


def _dense2(x1p, acc, w1d, w2d, fc1_w, fc1_b):
    return pl.pallas_call(
        _dense2_body,
        grid=(GRID1,),
        in_specs=[
            pl.BlockSpec((ROW_BLK, D), lambda i: (i, 0)),
            pl.BlockSpec((NC, ROW_BLK, D), lambda i: (0, i, 0)),
            pl.BlockSpec((D, 20), lambda i: (0, 0)),
            pl.BlockSpec((D, 20), lambda i: (0, 0)),
            pl.BlockSpec((20, 10), lambda i: (0, 0)),
            pl.BlockSpec((1, 10), lambda i: (0, 0)),
        ],
        out_specs=pl.BlockSpec((ROW_BLK, 10), lambda i: (i, 0)),
        out_shape=jax.ShapeDtypeStruct((RT, 10), jnp.float32),
    )(x1p, acc, w1d, w2d, fc1_w, fc1_b)


def _fc2_body(x_ref, w_ref, b_ref, out_ref):
    i = pl.program_id(0)

    @pl.when(i == 0)
    def _init():
        out_ref[...] = jnp.zeros_like(out_ref)

    out_ref[...] += jnp.dot(x_ref[...], w_ref[...],
                            preferred_element_type=jnp.float32)

    @pl.when(i == GRIDF - 1)
    def _finish():
        z = out_ref[...] + b_ref[...]
        m = jnp.max(z, axis=1, keepdims=True)
        e = jnp.exp(z - m)
        out_ref[...] = e / jnp.sum(e, axis=1, keepdims=True)


def _fc2(x3r, fc2_wp, fc2_b):
    return pl.pallas_call(
        _fc2_body,
        grid=(GRIDF,),
        in_specs=[
            pl.BlockSpec((B, KB), lambda i: (0, i)),
            pl.BlockSpec((KB, 64), lambda i: (i, 0)),
            pl.BlockSpec((1, 64), lambda i: (0, 0)),
        ],
        out_specs=pl.BlockSpec((B, 64), lambda i: (0, 0)),
        out_shape=jax.ShapeDtypeStruct((B, 64), jnp.float32),
    )(x3r, fc2_wp, fc2_b)


@jax.jit
def kernel(verts, edges, w0a, b0a, w1a, b1a, w0b, b0b, w1b, b1b,
           fc1_w, fc1_b, fc2_w, fc2_b):
    f32 = jnp.float32

    ones = jnp.ones((V, 1), f32)
    vaug = jnp.concatenate([verts.astype(f32), ones], axis=1)
    vaug = jnp.pad(vaug, ((0, RT - V), (0, D - 4)))

    src = edges[:, 0]
    dst = edges[:, 1]
    padidx = (V + (jnp.arange(EP - E, dtype=jnp.int32) % 512)).astype(jnp.int32)
    srcp = jnp.concatenate([src, padidx])
    dstp = jnp.concatenate([dst, padidx])

    zeros_t = jnp.zeros((RT, D), f32)

    w1c = jnp.concatenate([
        jnp.concatenate([w0a, jnp.zeros((3, D - 5), f32)], axis=1),
        jnp.concatenate([b0a, jnp.array([1.0], f32),
                         jnp.zeros((D - 6,), f32)])[None, :],
        jnp.zeros((D - 4, D), f32),
    ], axis=0)
    w2c = jnp.concatenate([
        jnp.concatenate([w1a, jnp.zeros((3, D - 5), f32)], axis=1),
        jnp.concatenate([b1a, jnp.zeros((D - 5,), f32)])[None, :],
        jnp.zeros((D - 4, D), f32),
    ], axis=0)
    w1d = jnp.concatenate([w0b, b0b[None, :],
                           jnp.zeros((D - 6, 20), f32)], axis=0)
    w2d = jnp.concatenate([w1b, b1b[None, :],
                           jnp.zeros((D - 6, 20), f32)], axis=0)

    spmv = _make_spmv()
    acc1 = spmv(vaug, srcp, dstp, zeros_t)
    x1p = _dense1(vaug, acc1, w1c, w2c)

    acc2 = spmv(x1p, srcp, dstp, zeros_t)
    x3 = _dense2(x1p, acc2, w1d, w2d, fc1_w, fc1_b[None, :])

    x3r = x3[:V].reshape(B, VPM * 10)
    x3r = jnp.pad(x3r, ((0, 0), (0, KP - VPM * 10)))
    fc2_wp = jnp.pad(fc2_w, ((0, KP - VPM * 10), (0, 0)))
    return _fc2(x3r, fc2_wp, fc2_b[None, :])


# speedup vs baseline: 7.1570x; 1.3000x over previous
import functools

import jax
import jax.numpy as jnp
from jax import lax
from jax.experimental import pallas as pl
from jax.experimental.pallas import tpu as pltpu
from jax.experimental.pallas import tpu_sc as plsc

B = 16
VPM = 5850
V = B * VPM
E = 280704

NC = 2
NS = 16
NW = NC * NS

RT = 94208
D = 16
CH = 128
EP = 282624
EPW = EP // NW
CHUNKS = EPW // CH
RPT = RT // NS

ROW_BLK = 4096
GRID1 = RT // ROW_BLK

KP = 58880
KB = 1280
GRIDF = KP // KB


def _leaky_relu(x):
    return jnp.where(x >= 0, x, 0.01 * x)


NB = 3


@functools.lru_cache(maxsize=None)
def _make_spmv():
    mesh = plsc.VectorSubcoreMesh(core_axis_name="c", subcore_axis_name="s")

    @functools.partial(
        pl.kernel,
        out_type=jax.ShapeDtypeStruct((NC, RT, D), jnp.float32),
        mesh=mesh,
        compiler_params=pltpu.CompilerParams(use_tc_tiling_on_sc=False),
        scratch_types=[
            pltpu.VMEM_SHARED((RT, D), jnp.float32),
            pltpu.VMEM((CHUNKS, CH), jnp.int32),
            pltpu.VMEM((CHUNKS, CH), jnp.int32),
            [pltpu.VMEM((CH, D), jnp.float32) for _ in range(NB)],
            [pltpu.VMEM((CH, D), jnp.float32) for _ in range(NB)],
            [pltpu.SemaphoreType.DMA for _ in range(NB)],
            [pltpu.SemaphoreType.DMA for _ in range(NB)],
        ],
    )
    def spmv(table_hbm, src_hbm, dst_hbm, zeros_hbm, out_hbm,
             acc_sp, idx_s, idx_d, rows_d, rows_s, sem_d, sem_s):
        c = lax.axis_index("c")
        s = lax.axis_index("s")
        w = s * NC + c

        pltpu.sync_copy(src_hbm.at[w], idx_s)
        pltpu.sync_copy(dst_hbm.at[w], idx_d)
        pltpu.sync_copy(zeros_hbm.at[pl.ds(s * RPT, RPT)],
                        acc_sp.at[pl.ds(s * RPT, RPT)])
        plsc.subcore_barrier()

        def body(j, carry):
            base = j * NB
            cps = []
            for b in range(NB):
                i = base + b
                cps.append((
                    pltpu.async_copy(table_hbm.at[idx_d.at[i]], rows_d[b],
                                     sem_d[b]),
                    pltpu.async_copy(table_hbm.at[idx_s.at[i]], rows_s[b],
                                     sem_s[b]),
                ))
            for b in range(NB):
                i = base + b
                cps[b][0].wait()
                pltpu.sync_copy(rows_d[b], acc_sp.at[idx_s.at[i]], add=True)
                cps[b][1].wait()
                pltpu.sync_copy(rows_s[b], acc_sp.at[idx_d.at[i]], add=True)
            return carry

        lax.fori_loop(0, CHUNKS // NB, body, 0)
        plsc.subcore_barrier()
        pltpu.sync_copy(acc_sp.at[pl.ds(s * RPT, RPT)],
                        out_hbm.at[c, pl.ds(s * RPT, RPT)])

    return spmv


def _dense1_body(vaug_ref, acc_ref, w1_ref, w2_ref, out_ref):
    sfull = acc_ref[0] + acc_ref[1]
    x = jnp.dot(vaug_ref[...], w1_ref[...], preferred_element_type=jnp.float32)
    x += jnp.dot(sfull, w2_ref[...], preferred_element_type=jnp.float32)
    out_ref[...] = _leaky_relu(x)


def _dense1(vaug, acc, w1c, w2c):
    return pl.pallas_call(
        _dense1_body,
        grid=(GRID1,),
        in_specs=[
            pl.BlockSpec((ROW_BLK, D), lambda i: (i, 0)),
            pl.BlockSpec((NC, ROW_BLK, D), lambda i: (0, i, 0)),
            pl.BlockSpec((D, D), lambda i: (0, 0)),
            pl.BlockSpec((D, D), lambda i: (0, 0)),
        ],
        out_specs=pl.BlockSpec((ROW_BLK, D), lambda i: (i, 0)),
        out_shape=jax.ShapeDtypeStruct((RT, D), jnp.float32),
    )(vaug, acc, w1c, w2c)


def _dense2_body(x1_ref, acc_ref, w1_ref, w2_ref, fw_ref, fb_ref, out_ref):
    sfull = acc_ref[0] + acc_ref[1]
    x2 = jnp.dot(x1_ref[...], w1_ref[...], preferred_element_type=jnp.float32)
    x2 += jnp.dot(sfull, w2_ref[...], preferred_element_type=jnp.float32)
    x2 = _leaky_relu(x2)
    x3 = jnp.dot(x2, fw_ref[...], preferred_element_type=jnp.float32)
    x3 += fb_ref[...]
    out_ref[...] = _leaky_relu(x3)


def _dense2(x1p, acc, w1d, w2d, fc1_w, fc1_b):
    return pl.pallas_call(
        _dense2_body,
        grid=(GRID1,),
        in_specs=[
            pl.BlockSpec((ROW_BLK, D), lambda i: (i, 0)),
            pl.BlockSpec((NC, ROW_BLK, D), lambda i: (0, i, 0)),
            pl.BlockSpec((D, 20), lambda i: (0, 0)),
            pl.BlockSpec((D, 20), lambda i: (0, 0)),
            pl.BlockSpec((20, 10), lambda i: (0, 0)),
            pl.BlockSpec((1, 10), lambda i: (0, 0)),
        ],
        out_specs=pl.BlockSpec((ROW_BLK, 10), lambda i: (i, 0)),
        out_shape=jax.ShapeDtypeStruct((RT, 10), jnp.float32),
    )(x1p, acc, w1d, w2d, fc1_w, fc1_b)


def _fc2_body(x_ref, w_ref, b_ref, out_ref):
    i = pl.program_id(0)

    @pl.when(i == 0)
    def _init():
        out_ref[...] = jnp.zeros_like(out_ref)

    out_ref[...] += jnp.dot(x_ref[...], w_ref[...],
                            preferred_element_type=jnp.float32)

    @pl.when(i == GRIDF - 1)
    def _finish():
        z = out_ref[...] + b_ref[...]
        m = jnp.max(z, axis=1, keepdims=True)
        e = jnp.exp(z - m)
        out_ref[...] = e / jnp.sum(e, axis=1, keepdims=True)


def _fc2(x3r, fc2_wp, fc2_b):
    return pl.pallas_call(
        _fc2_body,
        grid=(GRIDF,),
        in_specs=[
            pl.BlockSpec((B, KB), lambda i: (0, i)),
            pl.BlockSpec((KB, 64), lambda i: (i, 0)),
            pl.BlockSpec((1, 64), lambda i: (0, 0)),
        ],
        out_specs=pl.BlockSpec((B, 64), lambda i: (0, 0)),
        out_shape=jax.ShapeDtypeStruct((B, 64), jnp.float32),
    )(x3r, fc2_wp, fc2_b)


@jax.jit
def kernel(verts, edges, w0a, b0a, w1a, b1a, w0b, b0b, w1b, b1b,
           fc1_w, fc1_b, fc2_w, fc2_b):
    f32 = jnp.float32

    ones = jnp.ones((V, 1), f32)
    vaug = jnp.concatenate([verts.astype(f32), ones], axis=1)
    vaug = jnp.pad(vaug, ((0, RT - V), (0, D - 4)))

    src = edges[:, 0]
    dst = edges[:, 1]
    padidx = (V + (jnp.arange(EP - E, dtype=jnp.int32) % 512)).astype(jnp.int32)
    srcp = jnp.concatenate([src, padidx]).reshape(NW, CHUNKS, CH)
    dstp = jnp.concatenate([dst, padidx]).reshape(NW, CHUNKS, CH)

    zeros_t = jnp.zeros((RT, D), f32)

    w1c = jnp.concatenate([
        jnp.concatenate([w0a, jnp.zeros((3, D - 5), f32)], axis=1),
        jnp.concatenate([b0a, jnp.array([1.0], f32),
                         jnp.zeros((D - 6,), f32)])[None, :],
        jnp.zeros((D - 4, D), f32),
    ], axis=0)
    w2c = jnp.concatenate([
        jnp.concatenate([w1a, jnp.zeros((3, D - 5), f32)], axis=1),
        jnp.concatenate([b1a, jnp.zeros((D - 5,), f32)])[None, :],
        jnp.zeros((D - 4, D), f32),
    ], axis=0)
    w1d = jnp.concatenate([w0b, b0b[None, :],
                           jnp.zeros((D - 6, 20), f32)], axis=0)
    w2d = jnp.concatenate([w1b, b1b[None, :],
                           jnp.zeros((D - 6, 20), f32)], axis=0)

    spmv = _make_spmv()
    acc1 = spmv(vaug, srcp, dstp, zeros_t)
    x1p = _dense1(vaug, acc1, w1c, w2c)

    acc2 = spmv(x1p, srcp, dstp, zeros_t)
    x3 = _dense2(x1p, acc2, w1d, w2d, fc1_w, fc1_b[None, :])

    x3r = x3[:V].reshape(B, VPM * 10)
    x3r = jnp.pad(x3r, ((0, 0), (0, KP - VPM * 10)))
    fc2_wp = jnp.pad(fc2_w, ((0, KP - VPM * 10), (0, 0)))
    return _fc2(x3r, fc2_wp, fc2_b[None, :])


# speedup vs baseline: 12.8622x; 1.7971x over previous
import functools

import jax
import jax.numpy as jnp
from jax import lax
from jax.experimental import pallas as pl
from jax.experimental.pallas import tpu as pltpu
from jax.experimental.pallas import tpu_sc as plsc

B = 16
VPM = 5850
V = B * VPM
E = 280704

NC = 2
NS = 16
NW = NC * NS

VPMP = 5888
RT = B * VPMP
D = 16
CH = 128
EP = 282624
EPW = EP // NW
CHUNKS = EPW // CH
RPT = RT // NS
NB = 3

PR = RT // 8
PRM = VPMP // 8
PBR = 16
GRIDF = PRM // PBR


def _leaky_relu(x):
    return jnp.where(x >= 0, x, 0.01 * x)


@functools.lru_cache(maxsize=None)
def _make_spmv():
    mesh = plsc.VectorSubcoreMesh(core_axis_name="c", subcore_axis_name="s")

    @functools.partial(
        pl.kernel,
        out_type=jax.ShapeDtypeStruct((NC, RT, D), jnp.float32),
        mesh=mesh,
        compiler_params=pltpu.CompilerParams(use_tc_tiling_on_sc=False),
        scratch_types=[
            pltpu.VMEM_SHARED((RT, D), jnp.float32),
            pltpu.VMEM((CHUNKS, CH), jnp.int32),
            pltpu.VMEM((CHUNKS, CH), jnp.int32),
            [pltpu.VMEM((CH, D), jnp.float32) for _ in range(NB)],
            [pltpu.VMEM((CH, D), jnp.float32) for _ in range(NB)],
            pltpu.VMEM((CH, D), jnp.float32),
            [pltpu.SemaphoreType.DMA for _ in range(NB)],
            [pltpu.SemaphoreType.DMA for _ in range(NB)],
        ],
    )
    def spmv(table_hbm, src_hbm, dst_hbm, out_hbm,
             acc_sp, idx_s, idx_d, rows_d, rows_s, ztile, sem_d, sem_s):
        c = lax.axis_index("c")
        s = lax.axis_index("s")
        w = s * NC + c

        pltpu.sync_copy(src_hbm.at[w], idx_s)
        pltpu.sync_copy(dst_hbm.at[w], idx_d)

        def zrow(i, carry):
            ztile[i, :] = jnp.zeros((D,), jnp.float32)
            return carry

        lax.fori_loop(0, CH, zrow, 0)

        def zcopy(k, carry):
            pltpu.sync_copy(ztile, acc_sp.at[pl.ds(s * RPT + k * CH, CH)])
            return carry

        lax.fori_loop(0, RPT // CH, zcopy, 0)
        plsc.subcore_barrier()

        def body(j, carry):
            base = j * NB
            cps = []
            for b in range(NB):
                i = base + b
                cps.append((
                    pltpu.async_copy(table_hbm.at[idx_d.at[i]], rows_d[b],
                                     sem_d[b]),
                    pltpu.async_copy(table_hbm.at[idx_s.at[i]], rows_s[b],
                                     sem_s[b]),
                ))
            for b in range(NB):
                i = base + b
                cps[b][0].wait()
                pltpu.sync_copy(rows_d[b], acc_sp.at[idx_s.at[i]], add=True)
                cps[b][1].wait()
                pltpu.sync_copy(rows_s[b], acc_sp.at[idx_d.at[i]], add=True)
            return carry

        lax.fori_loop(0, CHUNKS // NB, body, 0)
        plsc.subcore_barrier()
        pltpu.sync_copy(acc_sp.at[pl.ds(s * RPT, RPT)],
                        out_hbm.at[c, pl.ds(s * RPT, RPT)])

    return spmv


def _dense1_body(vaug_ref, acc_ref, w1_ref, w2_ref, out_ref):
    sfull = acc_ref[0] + acc_ref[1]
    x = jnp.dot(vaug_ref[...], w1_ref[...], preferred_element_type=jnp.float32)
    x += jnp.dot(sfull, w2_ref[...], preferred_element_type=jnp.float32)
    out_ref[...] = _leaky_relu(x)


def _dense1(vaug_pk, acc_pk, bdw1c, bdw2c):
    return pl.pallas_call(
        _dense1_body,
        grid=(B,),
        in_specs=[
            pl.BlockSpec((PRM, 128), lambda i: (i, 0)),
            pl.BlockSpec((NC, PRM, 128), lambda i: (0, i, 0)),
            pl.BlockSpec((128, 128), lambda i: (0, 0)),
            pl.BlockSpec((128, 128), lambda i: (0, 0)),
        ],
        out_specs=pl.BlockSpec((PRM, 128), lambda i: (i, 0)),
        out_shape=jax.ShapeDtypeStruct((PR, 128), jnp.float32),
    )(vaug_pk, acc_pk, bdw1c, bdw2c)


def _dense2_body(x1_ref, acc_ref, w1_ref, w2_ref, f_ref, fb_ref, out_ref):
    sfull = acc_ref[0] + acc_ref[1]
    x2 = jnp.dot(x1_ref[...], w1_ref[...], preferred_element_type=jnp.float32)
    x2 += jnp.dot(sfull, w2_ref[...], preferred_element_type=jnp.float32)
    x2 = _leaky_relu(x2)
    x3 = jnp.dot(x2, f_ref[...], preferred_element_type=jnp.float32)
    x3 += fb_ref[...]
    x3 = _leaky_relu(x3)
    r_iota = lax.broadcasted_iota(jnp.int32, (PRM, 128), 0)
    l_iota = lax.broadcasted_iota(jnp.int32, (PRM, 128), 1)
    u = r_iota * 8 + l_iota // D
    out_ref[...] = jnp.where(u < VPM, x3, 0.0)


def _dense2(x1_pk, acc_pk, bdw1d, bdw2d, bdf, fb_bcast):
    return pl.pallas_call(
        _dense2_body,
        grid=(B,),
        in_specs=[
            pl.BlockSpec((PRM, 128), lambda i: (i, 0)),
            pl.BlockSpec((NC, PRM, 128), lambda i: (0, i, 0)),
            pl.BlockSpec((128, 256), lambda i: (0, 0)),
            pl.BlockSpec((128, 256), lambda i: (0, 0)),
            pl.BlockSpec((256, 128), lambda i: (0, 0)),
            pl.BlockSpec((1, 128), lambda i: (0, 0)),
        ],
        out_specs=pl.BlockSpec((PRM, 128), lambda i: (i, 0)),
        out_shape=jax.ShapeDtypeStruct((PR, 128), jnp.float32),
    )(x1_pk, acc_pk, bdw1d, bdw2d, bdf, fb_bcast)


def _fc2_body(x_ref, w_ref, b_ref, out_ref):
    i = pl.program_id(0)

    @pl.when(i == 0)
    def _init():
        out_ref[...] = jnp.zeros_like(out_ref)

    part = lax.dot_general(
        x_ref[...], w_ref[...],
        dimension_numbers=(((2,), (1,)), ((1,), (0,))),
        preferred_element_type=jnp.float32,
    )
    out_ref[...] += jnp.sum(part, axis=0)

    @pl.when(i == GRIDF - 1)
    def _finish():
        z = out_ref[...] + b_ref[...]
        m = jnp.max(z, axis=1, keepdims=True)
        e = jnp.exp(z - m)
        out_ref[...] = e / jnp.sum(e, axis=1, keepdims=True)


def _fc2(x3v, w3v, fc2_b):
    return pl.pallas_call(
        _fc2_body,
        grid=(GRIDF,),
        in_specs=[
            pl.BlockSpec((B, PBR, 128), lambda i: (0, i, 0)),
            pl.BlockSpec((PBR, 128, 64), lambda i: (i, 0, 0)),
            pl.BlockSpec((1, 64), lambda i: (0, 0)),
        ],
        out_specs=pl.BlockSpec((B, 64), lambda i: (0, 0)),
        out_shape=jax.ShapeDtypeStruct((B, 64), jnp.float32),
    )(x3v, w3v, fc2_b)


@jax.jit
def kernel(verts, edges, w0a, b0a, w1a, b1a, w0b, b0b, w1b, b1b,
           fc1_w, fc1_b, fc2_w, fc2_b):
    f32 = jnp.float32
    eye8 = jnp.eye(8, dtype=f32)

    va = jnp.concatenate([verts.astype(f32), jnp.ones((V, 1), f32)], axis=1)
    va = va.reshape(B, VPM, 4)
    va = jnp.pad(va, ((0, 0), (0, VPMP - VPM), (0, D - 4)))
    vaug_pk = va.reshape(PR, 128)
    vaug_t = vaug_pk.reshape(RT, D)

    src = edges[:, 0]
    dst = edges[:, 1]
    src = src + 38 * (src // VPM)
    dst = dst + 38 * (dst // VPM)
    i = jnp.arange(EP - E, dtype=jnp.int32)
    padidx = (i % B) * VPMP + VPM + (i // B) % (VPMP - VPM)
    srcp = jnp.concatenate([src, padidx]).reshape(NW, CHUNKS, CH)
    dstp = jnp.concatenate([dst, padidx]).reshape(NW, CHUNKS, CH)

    w1c = jnp.zeros((D, D), f32)
    w1c = w1c.at[:3, :5].set(w0a).at[3, :5].set(b0a).at[3, 5].set(1.0)
    w2c = jnp.zeros((D, D), f32)
    w2c = w2c.at[:3, :5].set(w1a).at[3, :5].set(b1a)
    bdw1c = jnp.kron(eye8, w1c)
    bdw2c = jnp.kron(eye8, w2c)

    w1d = jnp.zeros((D, 32), f32)
    w1d = w1d.at[:5, :20].set(w0b).at[5, :20].set(b0b)
    w2d = jnp.zeros((D, 32), f32)
    w2d = w2d.at[:5, :20].set(w1b).at[5, :20].set(b1b)
    bdw1d = jnp.kron(eye8, w1d)
    bdw2d = jnp.kron(eye8, w2d)
    fpad = jnp.zeros((32, D), f32).at[:20, :10].set(fc1_w)
    bdf = jnp.kron(eye8, fpad)
    fb_bcast = jnp.tile(jnp.pad(fc1_b, (0, D - 10)), 8)[None, :]

    fw = fc2_w.reshape(VPM, 10, 64)
    fw = jnp.pad(fw, ((0, VPMP - VPM), (0, D - 10), (0, 0)))
    w3v = fw.reshape(PRM, 128, 64)

    spmv = _make_spmv()
    acc1 = spmv(vaug_t, srcp, dstp)
    acc1_pk = acc1.reshape(NC, PR, 128)
    x1_pk = _dense1(vaug_pk, acc1_pk, bdw1c, bdw2c)

    acc2 = spmv(x1_pk.reshape(RT, D), srcp, dstp)
    acc2_pk = acc2.reshape(NC, PR, 128)
    x3_pk = _dense2(x1_pk, acc2_pk, bdw1d, bdw2d, bdf, fb_bcast)

    x3v = x3_pk.reshape(B, PRM, 128)
    return _fc2(x3v, w3v, fc2_b[None, :])


# speedup vs baseline: 12.8766x; 1.0011x over previous
import functools

import jax
import jax.numpy as jnp
from jax import lax
from jax.experimental import pallas as pl
from jax.experimental.pallas import tpu as pltpu
from jax.experimental.pallas import tpu_sc as plsc

B = 16
VPM = 5850
V = B * VPM
E = 280704

NC = 2
NS = 16
NW = NC * NS

VPMP = 5888
RT = B * VPMP
D = 16
CH = 128
EP = 282624
EPW = EP // NW
CHUNKS = EPW // CH
RPT = RT // NS
NB = 3

PR = RT // 8
PRM = VPMP // 8
PBR = 16
GRIDF = PRM // PBR


def _leaky_relu(x):
    return jnp.where(x >= 0, x, 0.01 * x)


@functools.lru_cache(maxsize=None)
def _make_spmv():
    mesh = plsc.VectorSubcoreMesh(core_axis_name="c", subcore_axis_name="s")

    @functools.partial(
        pl.kernel,
        out_type=jax.ShapeDtypeStruct((NC, RT, D), jnp.float32),
        mesh=mesh,
        compiler_params=pltpu.CompilerParams(use_tc_tiling_on_sc=False,
                                             skip_device_barrier=True),
        scratch_types=[
            pltpu.VMEM_SHARED((RT, D), jnp.float32),
            pltpu.VMEM((CHUNKS, CH), jnp.int32),
            pltpu.VMEM((CHUNKS, CH), jnp.int32),
            [pltpu.VMEM((CH, D), jnp.float32) for _ in range(NB)],
            [pltpu.VMEM((CH, D), jnp.float32) for _ in range(NB)],
            pltpu.VMEM((CH, D), jnp.float32),
            [pltpu.SemaphoreType.DMA for _ in range(NB)],
            [pltpu.SemaphoreType.DMA for _ in range(NB)],
        ],
    )
    def spmv(table_hbm, src_hbm, dst_hbm, out_hbm,
             acc_sp, idx_s, idx_d, rows_d, rows_s, ztile, sem_d, sem_s):
        c = lax.axis_index("c")
        s = lax.axis_index("s")
        w = s * NC + c

        pltpu.sync_copy(src_hbm.at[w], idx_s)
        pltpu.sync_copy(dst_hbm.at[w], idx_d)

        def zrow(i, carry):
            ztile[i, :] = jnp.zeros((D,), jnp.float32)
            return carry

        lax.fori_loop(0, CH, zrow, 0)

        def zcopy(k, carry):
            pltpu.sync_copy(ztile, acc_sp.at[pl.ds(s * RPT + k * CH, CH)])
            return carry

        lax.fori_loop(0, RPT // CH, zcopy, 0)
        plsc.subcore_barrier()

        def body(j, carry):
            base = j * NB
            cps = []
            for b in range(NB):
                i = base + b
                cps.append((
                    pltpu.async_copy(table_hbm.at[idx_d.at[i]], rows_d[b],
                                     sem_d[b]),
                    pltpu.async_copy(table_hbm.at[idx_s.at[i]], rows_s[b],
                                     sem_s[b]),
                ))
            for b in range(NB):
                i = base + b
                cps[b][0].wait()
                pltpu.sync_copy(rows_d[b], acc_sp.at[idx_s.at[i]], add=True)
                cps[b][1].wait()
                pltpu.sync_copy(rows_s[b], acc_sp.at[idx_d.at[i]], add=True)
            return carry

        lax.fori_loop(0, CHUNKS // NB, body, 0)
        plsc.subcore_barrier()
        pltpu.sync_copy(acc_sp.at[pl.ds(s * RPT, RPT)],
                        out_hbm.at[c, pl.ds(s * RPT, RPT)])

    return spmv


def _dense1_body(vaug_ref, acc_ref, w1_ref, w2_ref, out_ref):
    sfull = acc_ref[0] + acc_ref[1]
    x = jnp.dot(vaug_ref[...], w1_ref[...], preferred_element_type=jnp.float32)
    x += jnp.dot(sfull, w2_ref[...], preferred_element_type=jnp.float32)
    out_ref[...] = _leaky_relu(x)


def _dense1(vaug_pk, acc_pk, bdw1c, bdw2c):
    return pl.pallas_call(
        _dense1_body,
        grid=(B,),
        in_specs=[
            pl.BlockSpec((PRM, 128), lambda i: (i, 0)),
            pl.BlockSpec((NC, PRM, 128), lambda i: (0, i, 0)),
            pl.BlockSpec((128, 128), lambda i: (0, 0)),
            pl.BlockSpec((128, 128), lambda i: (0, 0)),
        ],
        out_specs=pl.BlockSpec((PRM, 128), lambda i: (i, 0)),
        out_shape=jax.ShapeDtypeStruct((PR, 128), jnp.float32),
    )(vaug_pk, acc_pk, bdw1c, bdw2c)


def _dense2_body(x1_ref, acc_ref, w1_ref, w2_ref, f_ref, fb_ref, out_ref):
    sfull = acc_ref[0] + acc_ref[1]
    x2 = jnp.dot(x1_ref[...], w1_ref[...], preferred_element_type=jnp.float32)
    x2 += jnp.dot(sfull, w2_ref[...], preferred_element_type=jnp.float32)
    x2 = _leaky_relu(x2)
    x3 = jnp.dot(x2, f_ref[...], preferred_element_type=jnp.float32)
    x3 += fb_ref[...]
    x3 = _leaky_relu(x3)
    r_iota = lax.broadcasted_iota(jnp.int32, (PRM, 128), 0)
    l_iota = lax.broadcasted_iota(jnp.int32, (PRM, 128), 1)
    u = r_iota * 8 + l_iota // D
    out_ref[...] = jnp.where(u < VPM, x3, 0.0)


def _dense2(x1_pk, acc_pk, bdw1d, bdw2d, bdf, fb_bcast):
    return pl.pallas_call(
        _dense2_body,
        grid=(B,),
        in_specs=[
            pl.BlockSpec((PRM, 128), lambda i: (i, 0)),
            pl.BlockSpec((NC, PRM, 128), lambda i: (0, i, 0)),
            pl.BlockSpec((128, 256), lambda i: (0, 0)),
            pl.BlockSpec((128, 256), lambda i: (0, 0)),
            pl.BlockSpec((256, 128), lambda i: (0, 0)),
            pl.BlockSpec((1, 128), lambda i: (0, 0)),
        ],
        out_specs=pl.BlockSpec((PRM, 128), lambda i: (i, 0)),
        out_shape=jax.ShapeDtypeStruct((PR, 128), jnp.float32),
    )(x1_pk, acc_pk, bdw1d, bdw2d, bdf, fb_bcast)


def _fc2_body(x_ref, w_ref, b_ref, out_ref):
    i = pl.program_id(0)

    @pl.when(i == 0)
    def _init():
        out_ref[...] = jnp.zeros_like(out_ref)

    part = lax.dot_general(
        x_ref[...], w_ref[...],
        dimension_numbers=(((2,), (1,)), ((1,), (0,))),
        preferred_element_type=jnp.float32,
    )
    out_ref[...] += jnp.sum(part, axis=0)

    @pl.when(i == GRIDF - 1)
    def _finish():
        z = out_ref[...] + b_ref[...]
        m = jnp.max(z, axis=1, keepdims=True)
        e = jnp.exp(z - m)
        out_ref[...] = e / jnp.sum(e, axis=1, keepdims=True)


def _fc2(x3v, w3v, fc2_b):
    return pl.pallas_call(
        _fc2_body,
        grid=(GRIDF,),
        in_specs=[
            pl.BlockSpec((B, PBR, 128), lambda i: (0, i, 0)),
            pl.BlockSpec((PBR, 128, 64), lambda i: (i, 0, 0)),
            pl.BlockSpec((1, 64), lambda i: (0, 0)),
        ],
        out_specs=pl.BlockSpec((B, 64), lambda i: (0, 0)),
        out_shape=jax.ShapeDtypeStruct((B, 64), jnp.float32),
    )(x3v, w3v, fc2_b)


@jax.jit
def kernel(verts, edges, w0a, b0a, w1a, b1a, w0b, b0b, w1b, b1b,
           fc1_w, fc1_b, fc2_w, fc2_b):
    f32 = jnp.float32
    eye8 = jnp.eye(8, dtype=f32)

    va = jnp.concatenate([verts.astype(f32), jnp.ones((V, 1), f32)], axis=1)
    va = va.reshape(B, VPM, 4)
    va = jnp.pad(va, ((0, 0), (0, VPMP - VPM), (0, D - 4)))
    vaug_pk = va.reshape(PR, 128)
    vaug_t = vaug_pk.reshape(RT, D)

    src = edges[:, 0]
    dst = edges[:, 1]
    src = src + 38 * (src // VPM)
    dst = dst + 38 * (dst // VPM)
    i = jnp.arange(EP - E, dtype=jnp.int32)
    padidx = (i % B) * VPMP + VPM + (i // B) % (VPMP - VPM)
    srcp = jnp.concatenate([src, padidx]).reshape(NW, CHUNKS, CH)
    dstp = jnp.concatenate([dst, padidx]).reshape(NW, CHUNKS, CH)

    w1c = jnp.zeros((D, D), f32)
    w1c = w1c.at[:3, :5].set(w0a).at[3, :5].set(b0a).at[3, 5].set(1.0)
    w2c = jnp.zeros((D, D), f32)
    w2c = w2c.at[:3, :5].set(w1a).at[3, :5].set(b1a)
    bdw1c = jnp.kron(eye8, w1c)
    bdw2c = jnp.kron(eye8, w2c)

    w1d = jnp.zeros((D, 32), f32)
    w1d = w1d.at[:5, :20].set(w0b).at[5, :20].set(b0b)
    w2d = jnp.zeros((D, 32), f32)
    w2d = w2d.at[:5, :20].set(w1b).at[5, :20].set(b1b)
    bdw1d = jnp.kron(eye8, w1d)
    bdw2d = jnp.kron(eye8, w2d)
    fpad = jnp.zeros((32, D), f32).at[:20, :10].set(fc1_w)
    bdf = jnp.kron(eye8, fpad)
    fb_bcast = jnp.tile(jnp.pad(fc1_b, (0, D - 10)), 8)[None, :]

    fw = fc2_w.reshape(VPM, 10, 64)
    fw = jnp.pad(fw, ((0, VPMP - VPM), (0, D - 10), (0, 0)))
    w3v = fw.reshape(PRM, 128, 64)

    spmv = _make_spmv()
    acc1 = spmv(vaug_t, srcp, dstp)
    acc1_pk = acc1.reshape(NC, PR, 128)
    x1_pk = _dense1(vaug_pk, acc1_pk, bdw1c, bdw2c)

    acc2 = spmv(x1_pk.reshape(RT, D), srcp, dstp)
    acc2_pk = acc2.reshape(NC, PR, 128)
    x3_pk = _dense2(x1_pk, acc2_pk, bdw1d, bdw2d, bdf, fb_bcast)

    x3v = x3_pk.reshape(B, PRM, 128)
    return _fc2(x3v, w3v, fc2_b[None, :])
